# baseline (speedup 1.0000x reference)
import jax
import jax.numpy as jnp
from jax import lax
from jax.experimental import pallas as pl
from jax.experimental.pallas import tpu as pltpu
from jax.experimental.pallas import tpu_sc as plsc

N = 10000
C = 128
K = 32
DC = 8
NC = 2
NS = 16
NW = NC * NS
ROWS_W = 320
CH = 4
R = ROWS_W // CH
N_PAD = NW * ROWS_W
NBUF = 3


def _sc_body(feat_hbm, c8_hbm, idx_hbm, sumsf_hbm, sumsc_hbm,
             idx_v, accf_v, accc_v, feat_sh, c8_sh, semg, semi):
    sid = lax.axis_index("s")
    wid = sid * NC + lax.axis_index("c")
    rows16 = N // NS
    sl = pl.ds(sid * rows16, rows16)
    pltpu.sync_copy(feat_hbm.at[sl], feat_sh.at[sl])
    pltpu.sync_copy(c8_hbm.at[sl], c8_sh.at[sl])
    pltpu.sync_copy(idx_hbm.at[wid], idx_v)
    plsc.subcore_barrier()

    def _drain_wb(c):
        buf = c % NBUF

        def _drain(k, carry):
            row = k * CH + c
            pltpu.make_async_copy(feat_sh.at[idx_v.at[row]],
                                  accf_v.at[buf], semg).wait()
            pltpu.make_async_copy(c8_sh.at[idx_v.at[row]],
                                  accc_v.at[buf], semg).wait()
            return carry

        lax.fori_loop(1, K, _drain, 0)
        base = wid * ROWS_W + c * R
        pltpu.sync_copy(accf_v.at[buf], sumsf_hbm.at[pl.ds(base, R)])
        pltpu.sync_copy(accc_v.at[buf], sumsc_hbm.at[pl.ds(base, R)])

    for c in range(CH):
        buf = c % NBUF
        if c >= NBUF:
            _drain_wb(c - NBUF)
        f0 = pltpu.async_copy(feat_sh.at[idx_v.at[c]], accf_v.at[buf], semi)
        pltpu.async_copy(c8_sh.at[idx_v.at[c]], accc_v.at[buf], semi)
        f0.wait()
        pltpu.make_async_copy(c8_sh.at[idx_v.at[c]], accc_v.at[buf],
                              semi).wait()

        def _fire(k, carry):
            row = k * CH + c
            pltpu.async_copy(feat_sh.at[idx_v.at[row]], accf_v.at[buf],
                             semg, add=True)
            pltpu.async_copy(c8_sh.at[idx_v.at[row]], accc_v.at[buf],
                             semg, add=True)
            return carry

        lax.fori_loop(1, K, _fire, 0)

    for c in range(CH - NBUF, CH):
        _drain_wb(c)


def _sc_gather_sums(feat, c8, idx_r):
    mesh = plsc.VectorSubcoreMesh(core_axis_name="c", subcore_axis_name="s")
    return pl.kernel(
        _sc_body,
        out_type=(jax.ShapeDtypeStruct((N_PAD, C), jnp.bfloat16),
                  jax.ShapeDtypeStruct((N_PAD, DC), jnp.float32)),
        mesh=mesh,
        scratch_types=[
            pltpu.VMEM((K * CH, R), jnp.int32),
            pltpu.VMEM((NBUF, R, C), jnp.bfloat16),
            pltpu.VMEM((NBUF, R, DC), jnp.float32),
            pltpu.VMEM_SHARED((N, C), jnp.bfloat16),
            pltpu.VMEM_SHARED((N, DC), jnp.float32),
            pltpu.SemaphoreType.DMA,
            pltpu.SemaphoreType.DMA,
        ],
        compiler_params=pltpu.CompilerParams(use_tc_tiling_on_sc=False),
    )(feat, c8, idx_r)


def _tc1_body(feat_ref, w_ref, b_ref, out_ref):
    w1 = w_ref[...][:C].astype(jnp.bfloat16)
    out_ref[...] = (jnp.dot(feat_ref[...], w1,
                            preferred_element_type=jnp.float32)
                    + b_ref[...])


def _tc1(feat, w, b):
    br = 1000
    return pl.pallas_call(
        _tc1_body,
        grid=(N // br,),
        in_specs=[
            pl.BlockSpec((br, C), lambda i: (i, 0)),
            pl.BlockSpec((2 * C + 6, C), lambda i: (0, 0)),
            pl.BlockSpec((1, C), lambda i: (0, 0)),
        ],
        out_specs=pl.BlockSpec((br, C), lambda i: (i, 0)),
        out_shape=jax.ShapeDtypeStruct((N, C), jnp.float32),
    )(feat, w, b)


def _tc2_body(y1_ref, sumsf_ref, sumsc_ref, c8_ref, w_ref, out_ref):
    s = sumsf_ref[...]
    sc_ = sumsc_ref[...]
    c8 = c8_ref[...]
    w = w_ref[...]
    inv = jnp.float32(1.0 / K)
    m1 = sc_[:, 0:3] * inv
    m2 = sc_[:, 3:6] * inv
    rm = m1 - c8[:, 0:3]
    rs = jnp.sqrt(jnp.maximum(m2 - m1 * m1, 0.0))
    rel = jnp.concatenate([rm, rs], axis=1)
    y = (y1_ref[...]
         + jnp.dot(s, w[C:2 * C].astype(jnp.bfloat16),
                   preferred_element_type=jnp.float32) * inv
         + jnp.dot(rel, w[2 * C:2 * C + 6],
                   preferred_element_type=jnp.float32))
    out_ref[...] = y * jax.nn.sigmoid(y)


def _tc2(y1, sumsf, sumsc, c8, w):
    br = 1000
    return pl.pallas_call(
        _tc2_body,
        grid=(N // br,),
        in_specs=[
            pl.BlockSpec((br, C), lambda i: (i, 0)),
            pl.BlockSpec((br, C), lambda i: (i, 0)),
            pl.BlockSpec((br, DC), lambda i: (i, 0)),
            pl.BlockSpec((br, DC), lambda i: (i, 0)),
            pl.BlockSpec((2 * C + 6, C), lambda i: (0, 0)),
        ],
        out_specs=pl.BlockSpec((br, C), lambda i: (i, 0)),
        out_shape=jax.ShapeDtypeStruct((N, C), jnp.float32),
    )(y1, sumsf, sumsc, c8, w)


def kernel(feat, coords, knn_idx, W, b):
    feat = feat.astype(jnp.float32)
    coords = coords.astype(jnp.float32)
    idx32 = knn_idx.astype(jnp.int32)

    c8 = jnp.concatenate(
        [coords, coords * coords, jnp.zeros((N, DC - 6), jnp.float32)],
        axis=1)

    idx_pad = jnp.pad(idx32, ((0, N_PAD - N), (0, 0)))
    idx_r = (idx_pad.reshape(NW, CH, R, K)
             .transpose(0, 3, 1, 2)
             .reshape(NW, K * CH, R))

    featb = feat.astype(jnp.bfloat16)
    wf = W.astype(jnp.float32)
    y1 = _tc1(featb, wf, b.astype(jnp.float32).reshape(1, C))
    sumsf, sumsc = _sc_gather_sums(featb, c8, idx_r)
    return _tc2(y1, sumsf, sumsc, c8, wf)
